# baseline (speedup 1.0000x reference)
import functools

import jax
import jax.numpy as jnp
from jax import lax
from jax.experimental import pallas as pl
from jax.experimental.pallas import tpu as pltpu
from jax.experimental.pallas import tpu_sc as plsc

BATCH = 4096
EMBED_DIM = 32
V_USER = 1000000
NUM_CORES = 2
NUM_SUBCORES = 16
NUM_WORKERS = NUM_CORES * NUM_SUBCORES
B_PER_W = BATCH // NUM_WORKERS
L = 16
BLK = 128
TAIL_START = (V_USER // BLK) * BLK
TAIL = V_USER - TAIL_START
LAST_BLOCK = TAIL_START // BLK - 1
K = 8


def _make_user_tc_kernel():
    def make_index_map(k):
        def index_map(i, idx_ref):
            return (0, jnp.minimum(idx_ref[i * K + k] >> 7, LAST_BLOCK))
        return index_map

    def body(idx_ref, *args):
        blks = args[:K]
        out_ref = args[K]
        i = pl.program_id(0)
        cat = jnp.concatenate([b[...] for b in blks], axis=1)
        pos = jnp.stack(
            [idx_ref[i * K + k] & (BLK - 1) for k in range(K)]
        ).reshape(K, 1) + (jnp.arange(K, dtype=jnp.int32) * BLK).reshape(K, 1)
        onehot = (
            lax.broadcasted_iota(jnp.int32, (K, K * BLK), 1) == pos
        ).astype(jnp.float32)
        vals = lax.dot_general(
            onehot, cat, (((1,), (1,)), ((), ())),
            precision=lax.Precision.HIGHEST,
            preferred_element_type=jnp.float32,
        )
        out_ref[pl.ds(i * K, K), :] = vals

    return pl.pallas_call(
        body,
        grid_spec=pltpu.PrefetchScalarGridSpec(
            num_scalar_prefetch=1,
            grid=(BATCH // K,),
            in_specs=[
                pl.BlockSpec((EMBED_DIM, BLK), make_index_map(k))
                for k in range(K)
            ],
            out_specs=pl.BlockSpec(
                (BATCH, EMBED_DIM), lambda i, idx_ref: (0, 0)),
        ),
        out_shape=jax.ShapeDtypeStruct((BATCH, EMBED_DIM), jnp.float32),
    )


def _make_sc_kernel():
    mesh = plsc.VectorSubcoreMesh(core_axis_name="c", subcore_axis_name="s")
    out = jax.ShapeDtypeStruct((BATCH, EMBED_DIM), jnp.float32)

    @functools.partial(
        pl.kernel,
        mesh=mesh,
        out_type=(out, out, out),
        compiler_params=pltpu.CompilerParams(use_tc_tiling_on_sc=False),
        scratch_types=[
            pltpu.VMEM((B_PER_W,), jnp.int32),
            pltpu.VMEM((B_PER_W,), jnp.int32),
            pltpu.VMEM((B_PER_W,), jnp.int32),
            pltpu.VMEM((B_PER_W, EMBED_DIM), jnp.float32),
            pltpu.VMEM((B_PER_W, EMBED_DIM), jnp.float32),
            pltpu.VMEM((B_PER_W, EMBED_DIM), jnp.float32),
            pltpu.SemaphoreType.DMA,
        ],
    )
    def lookup(iid_hbm, cid_hbm, uid_hbm, wi_hbm, wc_hbm, wt_hbm,
               out_i, out_c, out_t,
               idx_i, idx_c, idx_t, rows_i, rows_c, rows_t, sem):
        wid = lax.axis_index("s") * NUM_CORES + lax.axis_index("c")
        base = wid * B_PER_W
        pltpu.sync_copy(iid_hbm.at[pl.ds(base, B_PER_W)], idx_i)
        pltpu.sync_copy(cid_hbm.at[pl.ds(base, B_PER_W)], idx_c)
        pltpu.sync_copy(uid_hbm.at[pl.ds(base, B_PER_W)], idx_t)
        for g in range(B_PER_W // L):
            sl = pl.ds(g * L, L)
            v = idx_t[sl] - TAIL_START
            v = lax.max(v, jnp.zeros((L,), jnp.int32))
            idx_t[sl] = lax.min(v, jnp.full((L,), TAIL - 1, jnp.int32))
        ci = pltpu.async_copy(wi_hbm.at[idx_i], rows_i, sem)
        cc = pltpu.async_copy(wc_hbm.at[idx_c], rows_c, sem)
        ct = pltpu.async_copy(wt_hbm.at[idx_t], rows_t, sem)
        ci.wait()
        cc.wait()
        ct.wait()
        pltpu.sync_copy(rows_i, out_i.at[pl.ds(base, B_PER_W)])
        pltpu.sync_copy(rows_c, out_c.at[pl.ds(base, B_PER_W)])
        pltpu.sync_copy(rows_t, out_t.at[pl.ds(base, B_PER_W)])

    return lookup


_user_tc = _make_user_tc_kernel()
_sc_lookup = _make_sc_kernel()


def kernel(user_id, item_id, category, W_user, W_item, W_category):
    uid = user_id.astype(jnp.int32)
    w_tail = lax.slice(W_user, (TAIL_START, 0), (V_USER, EMBED_DIM))
    out_i, out_c, out_t = _sc_lookup(
        item_id.astype(jnp.int32),
        category.astype(jnp.int32),
        uid,
        W_item,
        W_category,
        w_tail,
    )
    out_u = _user_tc(uid, *([W_user.T] * K))
    out_u = jnp.where((uid >= TAIL_START)[:, None], out_t, out_u)
    return (out_u, out_i, out_c)


# speedup vs baseline: 3.5389x; 3.5389x over previous
import functools

import jax
import jax.numpy as jnp
from jax import lax
from jax.experimental import pallas as pl
from jax.experimental.pallas import tpu as pltpu
from jax.experimental.pallas import tpu_sc as plsc

BATCH = 4096
EMBED_DIM = 32
NUM_CORES = 2
NUM_SUBCORES = 16
NUM_WORKERS = NUM_CORES * NUM_SUBCORES
B_PER_W = BATCH // NUM_WORKERS
L = 16
GROUPS = B_PER_W // L
NBUF = 4
BLK = 128


def _make_user_kernel():
    mesh = plsc.VectorSubcoreMesh(core_axis_name="c", subcore_axis_name="s")

    @functools.partial(
        pl.kernel,
        mesh=mesh,
        out_type=jax.ShapeDtypeStruct((EMBED_DIM, BATCH), jnp.float32),
        compiler_params=pltpu.CompilerParams(needs_layout_passes=False),
        scratch_types=[
            pltpu.VMEM((B_PER_W,), jnp.int32),
            pltpu.VMEM((NBUF, EMBED_DIM, BLK), jnp.float32),
            pltpu.VMEM((EMBED_DIM, B_PER_W), jnp.float32),
        ]
        + [pltpu.SemaphoreType.DMA] * NBUF,
    )
    def lookup(uid_hbm, w_hbm, out_hbm, idx_v, bounce, col_v, *sems):
        wid = lax.axis_index("s") * NUM_CORES + lax.axis_index("c")
        base = wid * B_PER_W
        pltpu.sync_copy(uid_hbm.at[pl.ds(base, B_PER_W)], idx_v)

        cvec0 = lax.iota(jnp.int32, L)
        cvec1 = cvec0 + L

        pending = []

        def extract(slot, j, roff):
            roffv = jnp.full((L,), roff, jnp.int32)
            jv = jnp.full((L,), j, jnp.int32)
            top = plsc.load_gather(bounce.at[slot], [cvec0, roffv])
            bot = plsc.load_gather(bounce.at[slot], [cvec1, roffv])
            plsc.store_scatter(col_v, [cvec0, jv], top)
            plsc.store_scatter(col_v, [cvec1, jv], bot)

        for g in range(GROUPS):
            rv = idx_v[pl.ds(g * L, L)]
            for l in range(L):
                j = g * L + l
                r = rv[l]
                rb = pl.multiple_of(lax.shift_left(
                    lax.shift_right_logical(r, 7), 7), BLK)
                roff = lax.bitwise_and(r, BLK - 1)
                slot = j % NBUF
                if len(pending) == NBUF:
                    cp, pslot, pj, proff = pending.pop(0)
                    cp.wait()
                    extract(pslot, pj, proff)
                cp = pltpu.async_copy(
                    w_hbm.at[:, pl.ds(rb, BLK)], bounce.at[slot], sems[slot]
                )
                pending.append((cp, slot, j, roff))
        for cp, pslot, pj, proff in pending:
            cp.wait()
            extract(pslot, pj, proff)

        pltpu.sync_copy(col_v, out_hbm.at[:, pl.ds(base, B_PER_W)])

    return lookup


def _make_small_tables_kernel():
    mesh = plsc.VectorSubcoreMesh(core_axis_name="c", subcore_axis_name="s")

    @functools.partial(
        pl.kernel,
        mesh=mesh,
        out_type=(
            jax.ShapeDtypeStruct((BATCH, EMBED_DIM), jnp.float32),
            jax.ShapeDtypeStruct((BATCH, EMBED_DIM), jnp.float32),
        ),
        compiler_params=pltpu.CompilerParams(use_tc_tiling_on_sc=False),
        scratch_types=[
            pltpu.VMEM((B_PER_W,), jnp.int32),
            pltpu.VMEM((B_PER_W,), jnp.int32),
            pltpu.VMEM((B_PER_W, EMBED_DIM), jnp.float32),
            pltpu.VMEM((B_PER_W, EMBED_DIM), jnp.float32),
            pltpu.SemaphoreType.DMA,
        ],
    )
    def lookup(iid_hbm, cid_hbm, wi_hbm, wc_hbm, out_i, out_c,
               idx_i, idx_c, rows_i, rows_c, sem):
        wid = lax.axis_index("s") * NUM_CORES + lax.axis_index("c")
        base = wid * B_PER_W
        pltpu.sync_copy(iid_hbm.at[pl.ds(base, B_PER_W)], idx_i)
        pltpu.sync_copy(cid_hbm.at[pl.ds(base, B_PER_W)], idx_c)
        ci = pltpu.async_copy(wi_hbm.at[idx_i], rows_i, sem)
        cc = pltpu.async_copy(wc_hbm.at[idx_c], rows_c, sem)
        ci.wait()
        cc.wait()
        pltpu.sync_copy(rows_i, out_i.at[pl.ds(base, B_PER_W)])
        pltpu.sync_copy(rows_c, out_c.at[pl.ds(base, B_PER_W)])

    return lookup


_user_lookup = _make_user_kernel()
_small_lookup = _make_small_tables_kernel()


def kernel(user_id, item_id, category, W_user, W_item, W_category):
    out_u = _user_lookup(user_id.astype(jnp.int32), W_user.T)
    out_i, out_c = _small_lookup(
        item_id.astype(jnp.int32),
        category.astype(jnp.int32),
        W_item,
        W_category,
    )
    return (out_u.T, out_i, out_c)
